# Initial kernel scaffold; baseline (speedup 1.0000x reference)
import jax
import jax.numpy as jnp
from jax.experimental import pallas as pl


def kernel(input, weight):
    raise NotImplementedError("write your pallas kernel here")


# speedup vs baseline: 11.4736x; 11.4736x over previous
import functools

import jax
import jax.numpy as jnp
from jax.experimental import pallas as pl
from jax.experimental.pallas import tpu as pltpu

Q = 1024
N = 100000
D = 16
K = 32
LANES = 128

NEG_INF = float("-inf")


def _topk_kernel(s, ck, qb, inp_ref, w_ref, vals_ref, flag_ref, r_ref):
    qstep = pl.program_id(0)
    step = pl.program_id(1)
    nsteps = pl.num_programs(1)

    @pl.when((qstep == 0) & (step == 0))
    def _init_flag():
        flag_ref[0, 0] = 0

    @pl.when(step == 0)
    def _init():
        r_ref[...] = jnp.full((s, qb, LANES), NEG_INF, jnp.float32)

    scores = jax.lax.dot_general(
        inp_ref[...], w_ref[...],
        dimension_numbers=(((1,), (1,)), ((), ())),
        preferred_element_type=jnp.float32,
    )

    def fold(masked):
        for j in range(ck // LANES):
            v = scores[:, j * LANES:(j + 1) * LANES]
            if masked:
                kidx = (step * ck + j * LANES
                        + jax.lax.broadcasted_iota(jnp.int32, (qb, LANES), 1))
                v = jnp.where(kidx < N, v, NEG_INF)
            for i in range(s):
                ri = r_ref[i]
                hi = jnp.maximum(ri, v)
                v = jnp.minimum(ri, v)
                r_ref[i] = hi

    @pl.when(step != nsteps - 1)
    def _fold_plain():
        fold(False)

    @pl.when(step == nsteps - 1)
    def _fold_masked():
        fold(True)

    @pl.when(step == nsteps - 1)
    def _finalize():
        cand = r_ref[...]
        pos = (jax.lax.broadcasted_iota(jnp.int32, (s, qb, LANES), 0) * LANES
               + jax.lax.broadcasted_iota(jnp.int32, (s, qb, LANES), 2))
        big = jnp.int32(s * LANES)
        cols = []
        for i in range(K):
            m = jnp.max(jnp.max(cand, axis=0), axis=1)
            cols.append(m[:, None])
            eq = cand == m[None, :, None]
            first = jnp.min(jnp.min(jnp.where(eq, pos, big), axis=0), axis=1)
            cand = jnp.where(pos == first[None, :, None], NEG_INF, cand)
        vals = jnp.concatenate(cols, axis=1)
        vals_ref[...] = vals
        tau = vals[:, K - 1][None, :, None]
        bad = jnp.any(r_ref[s - 1][None] > tau)
        flag_ref[0, 0] = jnp.maximum(flag_ref[0, 0], bad.astype(jnp.int32))


def _run(input, weight, s, ck, qb):
    n_pad = ((N + ck - 1) // ck) * ck
    nsteps = n_pad // ck
    wpad = jnp.pad(weight, ((0, n_pad - N), (0, 0)))
    vals, flag = pl.pallas_call(
        functools.partial(_topk_kernel, s, ck, qb),
        grid=(Q // qb, nsteps),
        in_specs=[
            pl.BlockSpec((qb, D), lambda q, i: (q, 0)),
            pl.BlockSpec((ck, D), lambda q, i: (i, 0)),
        ],
        out_specs=[
            pl.BlockSpec((qb, K), lambda q, i: (q, 0)),
            pl.BlockSpec(memory_space=pltpu.SMEM),
        ],
        out_shape=[
            jax.ShapeDtypeStruct((Q, K), jnp.float32),
            jax.ShapeDtypeStruct((1, 1), jnp.int32),
        ],
        scratch_shapes=[pltpu.VMEM((s, qb, LANES), jnp.float32)],
    )(input, wpad)
    return vals, flag


def kernel(input, weight):
    vals, flag = _run(input, weight, 8, 2048, 1024)

    def exact(_):
        v, _f = _run(input, weight, 32, 2048, 256)
        return v

    return jax.lax.cond(flag[0, 0] > 0, exact, lambda _: vals, None)


# speedup vs baseline: 13.3518x; 1.1637x over previous
import functools

import jax
import jax.numpy as jnp
from jax.experimental import pallas as pl
from jax.experimental.pallas import tpu as pltpu

Q = 1024
N = 100000
D = 16
K = 32
LANES = 128

NEG_INF = float("-inf")


def _topk_kernel(s, ck, qb, inp_ref, w_ref, vals_ref, flag_ref, r_ref):
    step = pl.program_id(1)
    nsteps = pl.num_programs(1)

    @pl.when(step == 0)
    def _init():
        flag_ref[0, 0, 0] = 0
        r_ref[...] = jnp.full((s, qb, LANES), NEG_INF, jnp.float32)

    scores = jax.lax.dot_general(
        inp_ref[...], w_ref[...],
        dimension_numbers=(((1,), (1,)), ((), ())),
        preferred_element_type=jnp.float32,
    )

    def fold(masked):
        for j in range(ck // LANES):
            v = scores[:, j * LANES:(j + 1) * LANES]
            if masked:
                kidx = (step * ck + j * LANES
                        + jax.lax.broadcasted_iota(jnp.int32, (qb, LANES), 1))
                v = jnp.where(kidx < N, v, NEG_INF)
            for i in range(s):
                ri = r_ref[i]
                hi = jnp.maximum(ri, v)
                v = jnp.minimum(ri, v)
                r_ref[i] = hi

    @pl.when(step != nsteps - 1)
    def _fold_plain():
        fold(False)

    @pl.when(step == nsteps - 1)
    def _fold_masked():
        fold(True)

    @pl.when(step == nsteps - 1)
    def _finalize():
        cand = r_ref[...]
        pos = (jax.lax.broadcasted_iota(jnp.int32, (s, qb, LANES), 0) * LANES
               + jax.lax.broadcasted_iota(jnp.int32, (s, qb, LANES), 2))
        big = jnp.int32(s * LANES)
        cols = []
        for i in range(K):
            m = jnp.max(jnp.max(cand, axis=0), axis=1)
            cols.append(m[:, None])
            eq = cand == m[None, :, None]
            first = jnp.min(jnp.min(jnp.where(eq, pos, big), axis=0), axis=1)
            cand = jnp.where(pos == first[None, :, None], NEG_INF, cand)
        vals = jnp.concatenate(cols, axis=1)
        vals_ref[...] = vals
        tau = vals[:, K - 1][None, :, None]
        bad = jnp.any(r_ref[s - 1][None] > tau)
        flag_ref[0, 0, 0] = bad.astype(jnp.int32)


def _run(input, weight, s, ck, qb):
    n_pad = ((N + ck - 1) // ck) * ck
    nsteps = n_pad // ck
    wpad = jnp.pad(weight, ((0, n_pad - N), (0, 0)))
    vals, flag = pl.pallas_call(
        functools.partial(_topk_kernel, s, ck, qb),
        grid=(Q // qb, nsteps),
        in_specs=[
            pl.BlockSpec((qb, D), lambda q, i: (q, 0)),
            pl.BlockSpec((ck, D), lambda q, i: (i, 0)),
        ],
        out_specs=[
            pl.BlockSpec((qb, K), lambda q, i: (q, 0)),
            pl.BlockSpec((1, 1, 1), lambda q, i: (q, 0, 0),
                         memory_space=pltpu.SMEM),
        ],
        out_shape=[
            jax.ShapeDtypeStruct((Q, K), jnp.float32),
            jax.ShapeDtypeStruct((Q // qb, 1, 1), jnp.int32),
        ],
        scratch_shapes=[pltpu.VMEM((s, qb, LANES), jnp.float32)],
        compiler_params=pltpu.CompilerParams(
            dimension_semantics=("parallel", "arbitrary")),
    )(input, wpad)
    return vals, flag


def kernel(input, weight):
    vals, flag = _run(input, weight, 8, 2048, 512)

    def exact(_):
        v, _f = _run(input, weight, 32, 2048, 256)
        return v

    return jax.lax.cond(jnp.any(flag > 0), exact, lambda _: vals, None)


# speedup vs baseline: 22.1407x; 1.6583x over previous
import functools

import jax
import jax.numpy as jnp
from jax.experimental import pallas as pl
from jax.experimental.pallas import tpu as pltpu

Q = 1024
N = 100000
D = 16
K = 32
LANES = 128

NEG_INF = float("-inf")

SORT8 = ((0, 1), (2, 3), (4, 5), (6, 7), (0, 2), (1, 3), (4, 6), (5, 7),
         (1, 2), (5, 6), (0, 4), (1, 5), (2, 6), (3, 7), (2, 4), (3, 5),
         (1, 2), (3, 4), (5, 6))
BITONIC8 = ((0, 4), (1, 5), (2, 6), (3, 7), (0, 2), (1, 3), (4, 6), (5, 7),
            (0, 1), (2, 3), (4, 5), (6, 7))


def _topk_kernel(s, ck, qb, inp_ref, w_ref, vals_ref, flag_ref, r_ref):
    step = pl.program_id(1)
    nsteps = pl.num_programs(1)

    @pl.when(step == 0)
    def _init():
        flag_ref[0, 0, 0] = 0
        r_ref[...] = jnp.full((s, qb, LANES), NEG_INF, jnp.float32)

    scores = jax.lax.dot_general(
        inp_ref[...], w_ref[...],
        dimension_numbers=(((1,), (1,)), ((), ())),
        preferred_element_type=jnp.float32,
    )

    def subvec(j, masked):
        v = scores[:, j * LANES:(j + 1) * LANES]
        if masked:
            kidx = (step * ck + j * LANES
                    + jax.lax.broadcasted_iota(jnp.int32, (qb, LANES), 1))
            v = jnp.where(kidx < N, v, NEG_INF)
        return v

    def fold(masked):
        if s == 8:
            for t in range(ck // LANES // 8):
                b = [subvec(8 * t + u, masked) for u in range(8)]
                for i, j in SORT8:
                    hi = jnp.maximum(b[i], b[j])
                    b[j] = jnp.minimum(b[i], b[j])
                    b[i] = hi
                ladder = [jnp.maximum(r_ref[i], b[7 - i]) for i in range(8)]
                for i, j in BITONIC8:
                    hi = jnp.maximum(ladder[i], ladder[j])
                    ladder[j] = jnp.minimum(ladder[i], ladder[j])
                    ladder[i] = hi
                for i in range(8):
                    r_ref[i] = ladder[i]
        else:
            for j in range(ck // LANES):
                v = subvec(j, masked)
                for i in range(s):
                    ri = r_ref[i]
                    hi = jnp.maximum(ri, v)
                    v = jnp.minimum(ri, v)
                    r_ref[i] = hi

    @pl.when(step != nsteps - 1)
    def _fold_plain():
        fold(False)

    @pl.when(step == nsteps - 1)
    def _fold_masked():
        fold(True)

    @pl.when(step == nsteps - 1)
    def _finalize():
        cand = r_ref[...]
        pos = (jax.lax.broadcasted_iota(jnp.int32, (s, qb, LANES), 0) * LANES
               + jax.lax.broadcasted_iota(jnp.int32, (s, qb, LANES), 2))
        big = jnp.int32(s * LANES)
        cols = []
        for i in range(K):
            m = jnp.max(jnp.max(cand, axis=0), axis=1)
            cols.append(m[:, None])
            eq = cand == m[None, :, None]
            first = jnp.min(jnp.min(jnp.where(eq, pos, big), axis=0), axis=1)
            cand = jnp.where(pos == first[None, :, None], NEG_INF, cand)
        vals = jnp.concatenate(cols, axis=1)
        vals_ref[...] = vals
        tau = vals[:, K - 1][None, :, None]
        bad = jnp.any(r_ref[s - 1][None] > tau)
        flag_ref[0, 0, 0] = bad.astype(jnp.int32)


def _run(input, weight, s, ck, qb):
    n_pad = ((N + ck - 1) // ck) * ck
    nsteps = n_pad // ck
    wpad = jnp.pad(weight, ((0, n_pad - N), (0, 0)))
    vals, flag = pl.pallas_call(
        functools.partial(_topk_kernel, s, ck, qb),
        grid=(Q // qb, nsteps),
        in_specs=[
            pl.BlockSpec((qb, D), lambda q, i: (q, 0)),
            pl.BlockSpec((ck, D), lambda q, i: (i, 0)),
        ],
        out_specs=[
            pl.BlockSpec((qb, K), lambda q, i: (q, 0)),
            pl.BlockSpec((1, 1, 1), lambda q, i: (q, 0, 0),
                         memory_space=pltpu.SMEM),
        ],
        out_shape=[
            jax.ShapeDtypeStruct((Q, K), jnp.float32),
            jax.ShapeDtypeStruct((Q // qb, 1, 1), jnp.int32),
        ],
        scratch_shapes=[pltpu.VMEM((s, qb, LANES), jnp.float32)],
        compiler_params=pltpu.CompilerParams(
            dimension_semantics=("parallel", "arbitrary")),
    )(input, wpad)
    return vals, flag


def kernel(input, weight):
    vals, flag = _run(input, weight, 8, 2048, 512)

    def exact(_):
        v, _f = _run(input, weight, 32, 2048, 256)
        return v

    return jax.lax.cond(jnp.any(flag > 0), exact, lambda _: vals, None)


# speedup vs baseline: 23.2027x; 1.0480x over previous
import functools

import jax
import jax.numpy as jnp
from jax.experimental import pallas as pl
from jax.experimental.pallas import tpu as pltpu

Q = 1024
N = 100000
D = 16
K = 32
LANES = 128

NEG_INF = float("-inf")

SORT8 = ((0, 1), (2, 3), (4, 5), (6, 7), (0, 2), (1, 3), (4, 6), (5, 7),
         (1, 2), (5, 6), (0, 4), (1, 5), (2, 6), (3, 7), (2, 4), (3, 5),
         (1, 2), (3, 4), (5, 6))
BITONIC8 = ((0, 4), (1, 5), (2, 6), (3, 7), (0, 2), (1, 3), (4, 6), (5, 7),
            (0, 1), (2, 3), (4, 5), (6, 7))


def _topk_kernel(s, ck, qb, inp_ref, w_ref, vals_ref, flag_ref, r_ref):
    step = pl.program_id(1)
    nsteps = pl.num_programs(1)

    @pl.when(step == 0)
    def _init():
        flag_ref[0, 0, 0] = 0
        r_ref[...] = jnp.full((s, qb, LANES), NEG_INF, jnp.float32)

    scores = jax.lax.dot_general(
        inp_ref[...], w_ref[...],
        dimension_numbers=(((1,), (1,)), ((), ())),
        preferred_element_type=jnp.float32,
    )

    def subvec(j, masked):
        v = scores[:, j * LANES:(j + 1) * LANES]
        if masked:
            kidx = (step * ck + j * LANES
                    + jax.lax.broadcasted_iota(jnp.int32, (qb, LANES), 1))
            v = jnp.where(kidx < N, v, NEG_INF)
        return v

    def fold(masked):
        if s == 8:
            for t in range(ck // LANES // 8):
                b = [subvec(8 * t + u, masked) for u in range(8)]
                for i, j in SORT8:
                    hi = jnp.maximum(b[i], b[j])
                    b[j] = jnp.minimum(b[i], b[j])
                    b[i] = hi
                ladder = [jnp.maximum(r_ref[i], b[7 - i]) for i in range(8)]
                for i, j in BITONIC8:
                    hi = jnp.maximum(ladder[i], ladder[j])
                    ladder[j] = jnp.minimum(ladder[i], ladder[j])
                    ladder[i] = hi
                for i in range(8):
                    r_ref[i] = ladder[i]
        else:
            for j in range(ck // LANES):
                v = subvec(j, masked)
                for i in range(s):
                    ri = r_ref[i]
                    hi = jnp.maximum(ri, v)
                    v = jnp.minimum(ri, v)
                    r_ref[i] = hi

    @pl.when(step != nsteps - 1)
    def _fold_plain():
        fold(False)

    @pl.when(step == nsteps - 1)
    def _fold_masked():
        fold(True)

    @pl.when(step == nsteps - 1)
    def _finalize():
        worst = r_ref[s - 1]
        lanepos = jax.lax.broadcasted_iota(jnp.int32, (qb, LANES), 1)
        planes = [r_ref[i] for i in range(s)]
        cols = []
        for i in range(K):
            top = planes[0]
            m = jnp.max(top, axis=1, keepdims=True)
            cols.append(m)
            eq = top == m
            firstlane = jnp.min(jnp.where(eq, lanepos, LANES), axis=1,
                                keepdims=True)
            mask = lanepos == firstlane
            for t in range(s - 1):
                planes[t] = jnp.where(mask, planes[t + 1], planes[t])
            planes[s - 1] = jnp.where(mask, NEG_INF, planes[s - 1])
        vals = jnp.concatenate(cols, axis=1)
        vals_ref[...] = vals
        tau = vals[:, K - 1][:, None]
        bad = jnp.any(worst > tau)
        flag_ref[0, 0, 0] = bad.astype(jnp.int32)


def _run(input, weight, s, ck, qb):
    n_pad = ((N + ck - 1) // ck) * ck
    nsteps = n_pad // ck
    wpad = jnp.pad(weight, ((0, n_pad - N), (0, 0)))
    vals, flag = pl.pallas_call(
        functools.partial(_topk_kernel, s, ck, qb),
        grid=(Q // qb, nsteps),
        in_specs=[
            pl.BlockSpec((qb, D), lambda q, i: (q, 0)),
            pl.BlockSpec((ck, D), lambda q, i: (i, 0)),
        ],
        out_specs=[
            pl.BlockSpec((qb, K), lambda q, i: (q, 0)),
            pl.BlockSpec((1, 1, 1), lambda q, i: (q, 0, 0),
                         memory_space=pltpu.SMEM),
        ],
        out_shape=[
            jax.ShapeDtypeStruct((Q, K), jnp.float32),
            jax.ShapeDtypeStruct((Q // qb, 1, 1), jnp.int32),
        ],
        scratch_shapes=[pltpu.VMEM((s, qb, LANES), jnp.float32)],
        compiler_params=pltpu.CompilerParams(
            dimension_semantics=("parallel", "arbitrary")),
    )(input, wpad)
    return vals, flag


def kernel(input, weight):
    vals, flag = _run(input, weight, 8, 2048, 512)

    def exact(_):
        v, _f = _run(input, weight, 32, 2048, 256)
        return v

    return jax.lax.cond(jnp.any(flag > 0), exact, lambda _: vals, None)


# speedup vs baseline: 23.7219x; 1.0224x over previous
import functools

import jax
import jax.numpy as jnp
from jax.experimental import pallas as pl
from jax.experimental.pallas import tpu as pltpu

Q = 1024
N = 100000
D = 16
K = 32
LANES = 128

NEG_INF = float("-inf")

SORT8 = ((0, 1), (2, 3), (4, 5), (6, 7), (0, 2), (1, 3), (4, 6), (5, 7),
         (1, 2), (5, 6), (0, 4), (1, 5), (2, 6), (3, 7), (2, 4), (3, 5),
         (1, 2), (3, 4), (5, 6))
BITONIC8 = ((0, 4), (1, 5), (2, 6), (3, 7), (0, 2), (1, 3), (4, 6), (5, 7),
            (0, 1), (2, 3), (4, 5), (6, 7))


def _topk_kernel(s, ck, qb, inp_ref, w_ref, vals_ref, flag_ref, r_ref):
    step = pl.program_id(1)
    nsteps = pl.num_programs(1)

    @pl.when(step == 0)
    def _init():
        flag_ref[0, 0, 0] = 0
        r_ref[...] = jnp.full((s, qb, LANES), NEG_INF, jnp.float32)

    scores = jax.lax.dot_general(
        inp_ref[...], w_ref[...],
        dimension_numbers=(((1,), (1,)), ((), ())),
        preferred_element_type=jnp.float32,
    )

    def subvec(j, masked):
        v = scores[:, j * LANES:(j + 1) * LANES]
        if masked:
            kidx = (step * ck + j * LANES
                    + jax.lax.broadcasted_iota(jnp.int32, (qb, LANES), 1))
            v = jnp.where(kidx < N, v, NEG_INF)
        return v

    def fold(masked):
        if s == 8:
            for t in range(ck // LANES // 8):
                b = [subvec(8 * t + u, masked) for u in range(8)]
                for i, j in SORT8:
                    hi = jnp.maximum(b[i], b[j])
                    b[j] = jnp.minimum(b[i], b[j])
                    b[i] = hi
                ladder = [jnp.maximum(r_ref[i], b[7 - i]) for i in range(8)]
                for i, j in BITONIC8:
                    hi = jnp.maximum(ladder[i], ladder[j])
                    ladder[j] = jnp.minimum(ladder[i], ladder[j])
                    ladder[i] = hi
                for i in range(8):
                    r_ref[i] = ladder[i]
        else:
            for j in range(ck // LANES):
                v = subvec(j, masked)
                for i in range(s):
                    ri = r_ref[i]
                    hi = jnp.maximum(ri, v)
                    v = jnp.minimum(ri, v)
                    r_ref[i] = hi

    @pl.when(step != nsteps - 1)
    def _fold_plain():
        fold(False)

    @pl.when(step == nsteps - 1)
    def _fold_masked():
        fold(True)

    @pl.when(step == nsteps - 1)
    def _finalize():
        worst = r_ref[s - 1]
        lanepos = jax.lax.broadcasted_iota(jnp.int32, (qb, LANES), 1)
        planes = [r_ref[i] for i in range(s)]
        cols = []
        for i in range(K):
            top = planes[0]
            m = jnp.max(top, axis=1, keepdims=True)
            cols.append(m)
            eq = top == m
            firstlane = jnp.min(jnp.where(eq, lanepos, LANES), axis=1,
                                keepdims=True)
            mask = lanepos == firstlane
            for t in range(s - 1):
                planes[t] = jnp.where(mask, planes[t + 1], planes[t])
            planes[s - 1] = jnp.where(mask, NEG_INF, planes[s - 1])
        vals = jnp.concatenate(cols, axis=1)
        vals_ref[...] = vals
        tau = vals[:, K - 1][:, None]
        bad = jnp.any(worst > tau)
        flag_ref[0, 0, 0] = bad.astype(jnp.int32)


def _run(input, weight, s, ck, qb):
    n_pad = ((N + ck - 1) // ck) * ck
    nsteps = n_pad // ck
    wpad = jnp.pad(weight, ((0, n_pad - N), (0, 0)))
    vals, flag = pl.pallas_call(
        functools.partial(_topk_kernel, s, ck, qb),
        grid=(Q // qb, nsteps),
        in_specs=[
            pl.BlockSpec((qb, D), lambda q, i: (q, 0)),
            pl.BlockSpec((ck, D), lambda q, i: (i, 0)),
        ],
        out_specs=[
            pl.BlockSpec((qb, K), lambda q, i: (q, 0)),
            pl.BlockSpec((1, 1, 1), lambda q, i: (q, 0, 0),
                         memory_space=pltpu.SMEM),
        ],
        out_shape=[
            jax.ShapeDtypeStruct((Q, K), jnp.float32),
            jax.ShapeDtypeStruct((Q // qb, 1, 1), jnp.int32),
        ],
        scratch_shapes=[pltpu.VMEM((s, qb, LANES), jnp.float32)],
        compiler_params=pltpu.CompilerParams(
            dimension_semantics=("parallel", "arbitrary")),
    )(input, wpad)
    return vals, flag


def kernel(input, weight):
    vals, flag = _run(input, weight, 8, 4096, 512)

    def exact(_):
        v, _f = _run(input, weight, 32, 2048, 256)
        return v

    return jax.lax.cond(jnp.any(flag > 0), exact, lambda _: vals, None)


# speedup vs baseline: 30.5762x; 1.2889x over previous
import functools

import jax
import jax.numpy as jnp
from jax.experimental import pallas as pl
from jax.experimental.pallas import tpu as pltpu

Q = 1024
N = 100000
D = 16
K = 32
LANES = 128

NEG_INF = float("-inf")

SORT8 = ((0, 1), (2, 3), (4, 5), (6, 7), (0, 2), (1, 3), (4, 6), (5, 7),
         (1, 2), (5, 6), (0, 4), (1, 5), (2, 6), (3, 7), (2, 4), (3, 5),
         (1, 2), (3, 4), (5, 6))
BITONIC8 = ((0, 4), (1, 5), (2, 6), (3, 7), (0, 2), (1, 3), (4, 6), (5, 7),
            (0, 1), (2, 3), (4, 5), (6, 7))


def _topk_kernel(s, ck, qb, inp_ref, w_ref, vals_ref, flag_ref, r_ref):
    step = pl.program_id(1)
    nsteps = pl.num_programs(1)

    @pl.when(step == 0)
    def _init():
        flag_ref[0, 0, 0] = 0
        r_ref[...] = jnp.full((s, qb, LANES), NEG_INF, jnp.float32)

    SUB = 8 * LANES

    def sub_dot(t):
        return jax.lax.dot_general(
            inp_ref[...], w_ref[pl.ds(t * SUB, SUB), :],
            dimension_numbers=(((1,), (1,)), ((), ())),
            preferred_element_type=jnp.float32,
        )

    def subvec(scores, t, u, masked):
        v = scores[:, u * LANES:(u + 1) * LANES]
        if masked:
            kidx = (step * ck + t * SUB + u * LANES
                    + jax.lax.broadcasted_iota(jnp.int32, (qb, LANES), 1))
            v = jnp.where(kidx < N, v, NEG_INF)
        return v

    def fold_block(scores, t, masked):
        if s == 8:
            b = [subvec(scores, t, u, masked) for u in range(8)]
            for i, j in SORT8:
                hi = jnp.maximum(b[i], b[j])
                b[j] = jnp.minimum(b[i], b[j])
                b[i] = hi
            ladder = [jnp.maximum(r_ref[i], b[7 - i]) for i in range(8)]
            for i, j in BITONIC8:
                hi = jnp.maximum(ladder[i], ladder[j])
                ladder[j] = jnp.minimum(ladder[i], ladder[j])
                ladder[i] = hi
            for i in range(8):
                r_ref[i] = ladder[i]
        else:
            for u in range(8):
                v = subvec(scores, t, u, masked)
                for i in range(s):
                    ri = r_ref[i]
                    hi = jnp.maximum(ri, v)
                    v = jnp.minimum(ri, v)
                    r_ref[i] = hi

    def fold(masked):
        prev = sub_dot(0)
        for t in range(1, ck // SUB):
            cur = sub_dot(t)
            fold_block(prev, t - 1, masked)
            prev = cur
        fold_block(prev, ck // SUB - 1, masked)

    @pl.when(step != nsteps - 1)
    def _fold_plain():
        fold(False)

    @pl.when(step == nsteps - 1)
    def _fold_masked():
        fold(True)

    @pl.when(step == nsteps - 1)
    def _finalize():
        worst = r_ref[s - 1]
        lanepos = jax.lax.broadcasted_iota(jnp.int32, (qb, LANES), 1)
        planes = [r_ref[i] for i in range(s)]
        cols = []
        for i in range(K):
            top = planes[0]
            m = jnp.max(top, axis=1, keepdims=True)
            cols.append(m)
            eq = top == m
            firstlane = jnp.min(jnp.where(eq, lanepos, LANES), axis=1,
                                keepdims=True)
            mask = lanepos == firstlane
            for t in range(s - 1):
                planes[t] = jnp.where(mask, planes[t + 1], planes[t])
            planes[s - 1] = jnp.where(mask, NEG_INF, planes[s - 1])
        vals = jnp.concatenate(cols, axis=1)
        vals_ref[...] = vals
        tau = vals[:, K - 1][:, None]
        bad = jnp.any(worst > tau)
        flag_ref[0, 0, 0] = bad.astype(jnp.int32)


def _run(input, weight, s, ck, qb):
    n_pad = ((N + ck - 1) // ck) * ck
    nsteps = n_pad // ck
    wpad = jnp.pad(weight, ((0, n_pad - N), (0, 0)))
    vals, flag = pl.pallas_call(
        functools.partial(_topk_kernel, s, ck, qb),
        grid=(Q // qb, nsteps),
        in_specs=[
            pl.BlockSpec((qb, D), lambda q, i: (q, 0)),
            pl.BlockSpec((ck, D), lambda q, i: (i, 0)),
        ],
        out_specs=[
            pl.BlockSpec((qb, K), lambda q, i: (q, 0)),
            pl.BlockSpec((1, 1, 1), lambda q, i: (q, 0, 0),
                         memory_space=pltpu.SMEM),
        ],
        out_shape=[
            jax.ShapeDtypeStruct((Q, K), jnp.float32),
            jax.ShapeDtypeStruct((Q // qb, 1, 1), jnp.int32),
        ],
        scratch_shapes=[pltpu.VMEM((s, qb, LANES), jnp.float32)],
        compiler_params=pltpu.CompilerParams(
            dimension_semantics=("parallel", "arbitrary")),
    )(input, wpad)
    return vals, flag


def kernel(input, weight):
    vals, flag = _run(input, weight, 8, 4096, 512)

    def exact(_):
        v, _f = _run(input, weight, 32, 2048, 256)
        return v

    return jax.lax.cond(jnp.any(flag > 0), exact, lambda _: vals, None)
